# baseline (speedup 1.0000x reference)
import dataclasses
import functools

import jax
import jax.numpy as jnp
from jax import lax
from jax.experimental import pallas as pl
from jax.experimental.pallas import tpu as pltpu
from jax.experimental.pallas import tpu_sc as plsc

N = 10000
E = 320000
NC, NS = 2, 16
NW = NC * NS
EPW = E // NW
CHUNK = 128
NCHUNK = 80
EPAD = NCHUNK * CHUNK - EPW
NDUMMY = 8
NHCH = EPW // 16
RPS = N // NS
NBUF = 8

F32 = jnp.float32


def _mesh():
    return plsc.VectorSubcoreMesh(
        core_axis_name="c", subcore_axis_name="s",
        num_cores=NC, num_subcores=NS)


def _sc_params():
    cp = pltpu.CompilerParams()
    fields = pltpu.CompilerParams.__dataclass_fields__
    if "needs_layout_passes" in fields:
        cp = dataclasses.replace(cp, needs_layout_passes=False)
    if "use_tc_tiling_on_sc" in fields:
        cp = dataclasses.replace(cp, use_tc_tiling_on_sc=False)
    return cp


def _hist(dst16):
    @functools.partial(
        pl.kernel,
        out_type=jax.ShapeDtypeStruct((NW, 1, N), F32),
        mesh=_mesh(),
        scratch_types=[
            pltpu.VMEM((N,), F32),
            pltpu.VMEM((NHCH, 16), jnp.int32),
        ],
        compiler_params=_sc_params(),
    )
    def k(dst_hbm, out_hbm, hist, idx):
        c = lax.axis_index("c")
        s = lax.axis_index("s")
        w = s * NC + c
        pltpu.sync_copy(dst_hbm.at[w], idx)

        @pl.loop(0, N, step=16)
        def _(i):
            hist.at[pl.ds(i, 16)][...] = jnp.zeros((16,), F32)

        ones = jnp.ones((16,), F32)

        @pl.loop(0, NHCH)
        def _(j):
            plsc.addupdate_scatter(hist, [idx.at[j][...]], ones)

        pltpu.sync_copy(hist, out_hbm.at[w, 0])

    return k(dst16)


def _agg(g, srcc, dstc, zeros, C, dt):
    @functools.partial(
        pl.kernel,
        out_type=jax.ShapeDtypeStruct((NC, NS, RPS, C), dt),
        mesh=_mesh(),
        scratch_types=[
            pltpu.VMEM((NCHUNK, CHUNK), jnp.int32),
            pltpu.VMEM((NCHUNK, CHUNK), jnp.int32),
        ] + [pltpu.VMEM((CHUNK, C), dt)] * NBUF + [
            pltpu.VMEM_SHARED((N + NDUMMY, C), dt),
        ] + [pltpu.SemaphoreType.DMA] * (2 * NBUF),
        compiler_params=_sc_params(),
    )
    def k(g_hbm, src_hbm, dst_hbm, z_hbm, out_hbm, srcv, dstv, *rest):
        bufs = rest[:NBUF]
        acc = rest[NBUF]
        gsems = rest[NBUF + 1:2 * NBUF + 1]
        ssems = rest[2 * NBUF + 1:]
        c = lax.axis_index("c")
        s = lax.axis_index("s")
        w = s * NC + c

        def start_g(j, b):
            pltpu.async_copy(g_hbm.at[srcv.at[j]], bufs[b], gsems[b])

        def wait_g(b):
            pltpu.make_async_copy(g_hbm.at[srcv.at[0]], bufs[b],
                                  gsems[b]).wait()

        def start_s(j, b):
            pltpu.async_copy(bufs[b], acc.at[dstv.at[j]], ssems[b], add=True)

        def wait_s(b):
            pltpu.make_async_copy(bufs[b], acc.at[dstv.at[0]],
                                  ssems[b]).wait()

        pltpu.sync_copy(src_hbm.at[w], srcv)
        pltpu.sync_copy(dst_hbm.at[w], dstv)
        for b in range(NBUF):
            start_g(b, b)
        r0 = s * RPS
        pltpu.sync_copy(z_hbm.at[s], acc.at[pl.ds(r0, RPS)])
        plsc.subcore_barrier()

        @pl.loop(0, NCHUNK - 2 * NBUF, step=NBUF)
        def _(k4):
            for b in range(NBUF):
                j = k4 + b
                wait_g(b)
                start_s(j, b)
                wait_s(b)
                start_g(j + NBUF, b)

        for b in range(NBUF):
            j = NCHUNK - 2 * NBUF + b
            wait_g(b)
            start_s(j, b)
            wait_s(b)
            start_g(j + NBUF, b)
        for b in range(NBUF):
            wait_g(b)
            start_s(NCHUNK - NBUF + b, b)
            wait_s(b)

        plsc.subcore_barrier()
        pltpu.sync_copy(acc.at[pl.ds(r0, RPS)], out_hbm.at[c, s])

    return k(g, srcc, dstc, zeros)


NCHUNK3 = NCHUNK * 2
NP = 10240
RPS3 = NP // NS


def _agg3solo(g3p, srcc3, dstc3, zeros, dinv3, b2d):
    C = 16

    @functools.partial(
        pl.kernel,
        out_type=jax.ShapeDtypeStruct((NS, RPS3, C), F32),
        mesh=_mesh(),
        scratch_types=[
            pltpu.VMEM((NCHUNK3, CHUNK), jnp.int32),
            pltpu.VMEM((NCHUNK3, CHUNK), jnp.int32),
        ] + [pltpu.VMEM((CHUNK, C), F32)] * NBUF + [
            pltpu.VMEM_SHARED((NP, C), F32),
            pltpu.VMEM((RPS3, C), F32),
            pltpu.VMEM((RPS3, C), F32),
            pltpu.VMEM((RPS3, C), F32),
            pltpu.VMEM((1, C), F32),
        ] + [pltpu.SemaphoreType.DMA] * (2 * NBUF),
        compiler_params=_sc_params(),
    )
    def k(g_hbm, src_hbm, dst_hbm, z_hbm, d_hbm, b_hbm, out_hbm,
          srcv, dstv, *rest):
        bufs = rest[:NBUF]
        acc = rest[NBUF]
        abuf, gbuf, dbuf, bbuf = rest[NBUF + 1:NBUF + 5]
        gsems = rest[NBUF + 5:2 * NBUF + 5]
        ssems = rest[2 * NBUF + 5:]
        c = lax.axis_index("c")
        s = lax.axis_index("s")

        def start_g(j, b):
            pltpu.async_copy(g_hbm.at[srcv.at[j]], bufs[b], gsems[b])

        def wait_g(b):
            pltpu.make_async_copy(g_hbm.at[srcv.at[0]], bufs[b],
                                  gsems[b]).wait()

        def start_s(j, b):
            pltpu.async_copy(bufs[b], acc.at[dstv.at[j]], ssems[b], add=True)

        def wait_s(b):
            pltpu.make_async_copy(bufs[b], acc.at[dstv.at[0]],
                                  ssems[b]).wait()

        @pl.when(c == 0)
        def _():
            pltpu.sync_copy(src_hbm.at[s], srcv)
            pltpu.sync_copy(dst_hbm.at[s], dstv)
            for b in range(NBUF):
                start_g(b, b)
            r0 = s * RPS3
            pltpu.sync_copy(z_hbm.at[s], acc.at[pl.ds(r0, RPS3)])
            plsc.subcore_barrier()

            @pl.loop(0, NCHUNK3 - 2 * NBUF, step=NBUF)
            def _(k4):
                for b in range(NBUF):
                    j = k4 + b
                    wait_g(b)
                    start_s(j, b)
                    wait_s(b)
                    start_g(j + NBUF, b)

            for b in range(NBUF):
                j = NCHUNK3 - 2 * NBUF + b
                wait_g(b)
                start_s(j, b)
                wait_s(b)
                start_g(j + NBUF, b)
            for b in range(NBUF):
                wait_g(b)
                start_s(NCHUNK3 - NBUF + b, b)
                wait_s(b)

            plsc.subcore_barrier()
            pltpu.sync_copy(acc.at[pl.ds(r0, RPS3)], abuf)
            pltpu.sync_copy(g_hbm.at[pl.ds(r0, RPS3)], gbuf)
            pltpu.sync_copy(d_hbm.at[s], dbuf)
            pltpu.sync_copy(b_hbm, bbuf)
            bv = bbuf.at[0][...]

            @pl.loop(0, RPS3)
            def _(i):
                abuf.at[i][...] = (abuf.at[i][...] + gbuf.at[i][...]) \
                    * dbuf.at[i][...] + bv

            pltpu.sync_copy(abuf, out_hbm.at[s])

    return k(g3p, srcc3, dstc3, zeros, dinv3, b2d).reshape(NP, C)[:N]


def _prep(x, W, counts):
    def body(x_ref, w_ref, c_ref, dinv_ref, gb_ref):
        deg = 1.0 + jnp.sum(c_ref[...], axis=0)
        dinv = lax.rsqrt(deg)[:, None]
        dinv_ref[...] = dinv
        g = jnp.dot(x_ref[...], w_ref[...],
                    preferred_element_type=F32) * dinv
        gb_ref[...] = g.astype(jnp.bfloat16)

    C = W.shape[1]
    return pl.pallas_call(
        body,
        out_shape=(jax.ShapeDtypeStruct((N, 1), F32),
                   jax.ShapeDtypeStruct((N, C), jnp.bfloat16)),
    )(x, W, counts)


def _layer(acc, g, dinv, b2d, W, out_dt):
    def body(a_ref, g_ref, d_ref, b_ref, w_ref, o_ref):
        a = (a_ref[0] + a_ref[1]).astype(F32)
        t = (a + g_ref[...].astype(F32)) * d_ref[...] + b_ref[...]
        z = jnp.maximum(t, 0.0)
        o = jnp.dot(z, w_ref[...], preferred_element_type=F32) * d_ref[...]
        o_ref[...] = o.astype(out_dt)

    C = W.shape[1]
    return pl.pallas_call(
        body,
        out_shape=jax.ShapeDtypeStruct((N, C), out_dt),
    )(acc, g, dinv, b2d, W)


def kernel(x, edge_index, W1, b1, W2, b2, W3, b3):
    src = edge_index[0].astype(jnp.int32)
    dst = edge_index[1].astype(jnp.int32)
    pad_src = jnp.broadcast_to((jnp.arange(EPAD, dtype=jnp.int32) * 41) % N,
                               (NW, EPAD))
    pad_dst = jnp.broadcast_to(N + (jnp.arange(EPAD, dtype=jnp.int32)
                                    % NDUMMY), (NW, EPAD))
    srcc = jnp.concatenate([src.reshape(NW, EPW), pad_src],
                           axis=1).reshape(NW, NCHUNK, CHUNK)
    dstc = jnp.concatenate([dst.reshape(NW, EPW), pad_dst],
                           axis=1).reshape(NW, NCHUNK, CHUNK)
    dst16 = dst.reshape(NW, NHCH, 16)
    BF16 = jnp.bfloat16
    z64 = jnp.zeros((NS, RPS, 64), BF16)

    counts = _hist(dst16).reshape(NW, N)
    dinv, g1 = _prep(x, W1, counts)
    acc1 = _agg(g1, srcc, dstc, z64, 64, BF16).reshape(NC, N, 64)
    g2 = _layer(acc1, g1, dinv, b1.reshape(1, -1), W2, BF16)
    acc2 = _agg(g2, srcc, dstc, z64, 64, BF16).reshape(NC, N, 64)
    g3 = _layer(acc2, g2, dinv, b2.reshape(1, -1), W3, F32)
    g3p = jnp.concatenate([g3, jnp.zeros((NP - N, 16), F32)], axis=0)
    dinvp = jnp.concatenate([dinv, jnp.ones((NP - N, 1), F32)], axis=0)
    dinv3 = jnp.broadcast_to(dinvp.reshape(NS, RPS3, 1), (NS, RPS3, 16))
    z16s = jnp.zeros((NS, RPS3, 16), F32)
    return _agg3solo(g3p, srcc.reshape(NS, NCHUNK3, CHUNK),
                     dstc.reshape(NS, NCHUNK3, CHUNK), z16s,
                     dinv3, b3.reshape(1, -1))


# speedup vs baseline: 1.0252x; 1.0252x over previous
import dataclasses
import functools

import jax
import jax.numpy as jnp
from jax import lax
from jax.experimental import pallas as pl
from jax.experimental.pallas import tpu as pltpu
from jax.experimental.pallas import tpu_sc as plsc

N = 10000
E = 320000
NC, NS = 2, 16
NW = NC * NS
EPW = E // NW
CHUNK = 128
NCHUNK = 80
EPAD = NCHUNK * CHUNK - EPW
NDUMMY = 8
NHCH = EPW // 16
RPS = N // NS
NBUF = 8

F32 = jnp.float32


def _mesh():
    return plsc.VectorSubcoreMesh(
        core_axis_name="c", subcore_axis_name="s",
        num_cores=NC, num_subcores=NS)


def _sc_params():
    cp = pltpu.CompilerParams()
    fields = pltpu.CompilerParams.__dataclass_fields__
    if "needs_layout_passes" in fields:
        cp = dataclasses.replace(cp, needs_layout_passes=False)
    if "use_tc_tiling_on_sc" in fields:
        cp = dataclasses.replace(cp, use_tc_tiling_on_sc=False)
    return cp


def _hist(dst16):
    @functools.partial(
        pl.kernel,
        out_type=jax.ShapeDtypeStruct((NW, 1, N), F32),
        mesh=_mesh(),
        scratch_types=[
            pltpu.VMEM((N,), F32),
            pltpu.VMEM((NHCH, 16), jnp.int32),
        ],
        compiler_params=_sc_params(),
    )
    def k(dst_hbm, out_hbm, hist, idx):
        c = lax.axis_index("c")
        s = lax.axis_index("s")
        w = s * NC + c
        pltpu.sync_copy(dst_hbm.at[w], idx)

        @pl.loop(0, N, step=16)
        def _(i):
            hist.at[pl.ds(i, 16)][...] = jnp.zeros((16,), F32)

        ones = jnp.ones((16,), F32)

        @pl.loop(0, NHCH)
        def _(j):
            plsc.addupdate_scatter(hist, [idx.at[j][...]], ones)

        pltpu.sync_copy(hist, out_hbm.at[w, 0])

    return k(dst16)


def _agg(g, srcc, dstc, zeros, C, dt):
    @functools.partial(
        pl.kernel,
        out_type=jax.ShapeDtypeStruct((NC, NS, RPS, C), dt),
        mesh=_mesh(),
        scratch_types=[
            pltpu.VMEM((NCHUNK, CHUNK), jnp.int32),
            pltpu.VMEM((NCHUNK, CHUNK), jnp.int32),
        ] + [pltpu.VMEM((CHUNK, C), dt)] * NBUF + [
            pltpu.VMEM_SHARED((N + NDUMMY, C), dt),
        ] + [pltpu.SemaphoreType.DMA] * (2 * NBUF),
        compiler_params=_sc_params(),
    )
    def k(g_hbm, src_hbm, dst_hbm, z_hbm, out_hbm, srcv, dstv, *rest):
        bufs = rest[:NBUF]
        acc = rest[NBUF]
        gsems = rest[NBUF + 1:2 * NBUF + 1]
        ssems = rest[2 * NBUF + 1:]
        c = lax.axis_index("c")
        s = lax.axis_index("s")
        w = s * NC + c

        def start_g(j, b):
            pltpu.async_copy(g_hbm.at[srcv.at[j]], bufs[b], gsems[b])

        def wait_g(b):
            pltpu.make_async_copy(g_hbm.at[srcv.at[0]], bufs[b],
                                  gsems[b]).wait()

        def start_s(j, b):
            pltpu.async_copy(bufs[b], acc.at[dstv.at[j]], ssems[b], add=True)

        def wait_s(b):
            pltpu.make_async_copy(bufs[b], acc.at[dstv.at[0]],
                                  ssems[b]).wait()

        pltpu.sync_copy(src_hbm.at[w], srcv)
        pltpu.sync_copy(dst_hbm.at[w], dstv)
        for b in range(NBUF):
            start_g(b, b)
        r0 = s * RPS
        pltpu.sync_copy(z_hbm.at[s], acc.at[pl.ds(r0, RPS)])
        plsc.subcore_barrier()

        @pl.loop(0, NCHUNK - 2 * NBUF, step=NBUF)
        def _(k4):
            for b in range(NBUF):
                j = k4 + b
                wait_g(b)
                start_s(j, b)
                wait_s(b)
                start_g(j + NBUF, b)

        for b in range(NBUF):
            j = NCHUNK - 2 * NBUF + b
            wait_g(b)
            start_s(j, b)
            wait_s(b)
            start_g(j + NBUF, b)
        for b in range(NBUF):
            wait_g(b)
            start_s(NCHUNK - NBUF + b, b)
            wait_s(b)

        plsc.subcore_barrier()
        pltpu.sync_copy(acc.at[pl.ds(r0, RPS)], out_hbm.at[c, s])

    return k(g, srcc, dstc, zeros)


def _prep(x, W, counts):
    def body(x_ref, w_ref, c_ref, dinv_ref, gb_ref):
        deg = 1.0 + jnp.sum(c_ref[...], axis=0)
        dinv = lax.rsqrt(deg)[:, None]
        dinv_ref[...] = dinv
        g = jnp.dot(x_ref[...], w_ref[...],
                    preferred_element_type=F32) * dinv
        gb_ref[...] = g.astype(jnp.bfloat16)

    C = W.shape[1]
    return pl.pallas_call(
        body,
        out_shape=(jax.ShapeDtypeStruct((N, 1), F32),
                   jax.ShapeDtypeStruct((N, C), jnp.bfloat16)),
    )(x, W, counts)


def _layer(acc, g, dinv, b2d, W, out_dt):
    def body(a_ref, g_ref, d_ref, b_ref, w_ref, o_ref):
        a = (a_ref[0] + a_ref[1]).astype(F32)
        t = (a + g_ref[...].astype(F32)) * d_ref[...] + b_ref[...]
        z = jnp.maximum(t, 0.0)
        o = jnp.dot(z, w_ref[...], preferred_element_type=F32) * d_ref[...]
        o_ref[...] = o.astype(out_dt)

    C = W.shape[1]
    return pl.pallas_call(
        body,
        out_shape=jax.ShapeDtypeStruct((N, C), out_dt),
    )(acc, g, dinv, b2d, W)


def _final(acc, g, dinv, b2d):
    def body(a_ref, g_ref, d_ref, b_ref, o_ref):
        a = (a_ref[0] + a_ref[1]).astype(F32)
        o_ref[...] = (a + g_ref[...]) * d_ref[...] + b_ref[...]

    return pl.pallas_call(
        body,
        out_shape=jax.ShapeDtypeStruct(g.shape, F32),
    )(acc, g, dinv, b2d)


def kernel(x, edge_index, W1, b1, W2, b2, W3, b3):
    src = edge_index[0].astype(jnp.int32)
    dst = edge_index[1].astype(jnp.int32)
    pad_src = jnp.broadcast_to((jnp.arange(EPAD, dtype=jnp.int32) * 41) % N,
                               (NW, EPAD))
    pad_dst = jnp.broadcast_to(N + (jnp.arange(EPAD, dtype=jnp.int32)
                                    % NDUMMY), (NW, EPAD))
    srcc = jnp.concatenate([src.reshape(NW, EPW), pad_src],
                           axis=1).reshape(NW, NCHUNK, CHUNK)
    dstc = jnp.concatenate([dst.reshape(NW, EPW), pad_dst],
                           axis=1).reshape(NW, NCHUNK, CHUNK)
    dst16 = dst.reshape(NW, NHCH, 16)
    BF16 = jnp.bfloat16
    z64 = jnp.zeros((NS, RPS, 64), BF16)
    z16 = jnp.zeros((NS, RPS, 16), F32)

    counts = _hist(dst16).reshape(NW, N)
    dinv, g1 = _prep(x, W1, counts)
    acc1 = _agg(g1, srcc, dstc, z64, 64, BF16).reshape(NC, N, 64)
    g2 = _layer(acc1, g1, dinv, b1.reshape(1, -1), W2, BF16)
    acc2 = _agg(g2, srcc, dstc, z64, 64, BF16).reshape(NC, N, 64)
    g3 = _layer(acc2, g2, dinv, b2.reshape(1, -1), W3, F32)
    acc3 = _agg(g3, srcc, dstc, z16, 16, F32).reshape(NC, N, 16)
    return _final(acc3, g3, dinv, b3.reshape(1, -1))
